# Initial kernel scaffold; baseline (speedup 1.0000x reference)
import jax
import jax.numpy as jnp
from jax.experimental import pallas as pl


def kernel(x, edge_index, W_enc, b_enc, W_lin0, W_self0, b_self0, W_lin1, W_self1, b_self1, Wn1, bn1, Wn2, bn2, Wc1, bc1, Wc2, bc2):
    raise NotImplementedError("write your pallas kernel here")


# speedup vs baseline: 5.5122x; 5.5122x over previous
import functools

import jax
import jax.numpy as jnp
from jax import lax
from jax.experimental import pallas as pl
from jax.experimental.pallas import tpu as pltpu
from jax.experimental.pallas import tpu_sc as plsc

N = 10000
NPAD = 10240
E = 320000
F = 128
H = 128
CHUNK = 128
NCORES = 2
NSUB = 16
NW = NCORES * NSUB
CHUNKS_PER_W = 79
EPAD = CHUNKS_PER_W * NW * CHUNK
ROWS_PER_SUB = NPAD // NSUB

BLK = 1024
GRID = NPAD // BLK

_sc_mesh = plsc.VectorSubcoreMesh(core_axis_name="c", subcore_axis_name="s")


def _sc_segment_sum(hw, src, dst, zeros):
    @functools.partial(
        pl.kernel,
        out_type=jax.ShapeDtypeStruct((NCORES, NPAD, H), jnp.float32),
        mesh=_sc_mesh,
        scratch_types=[
            pltpu.VMEM((CHUNK,), jnp.int32),
            pltpu.VMEM((CHUNK,), jnp.int32),
            pltpu.VMEM((CHUNK, H), jnp.float32),
            pltpu.VMEM_SHARED((NPAD, H), jnp.float32),
        ],
    )
    def seg_sum_kernel(hw_hbm, src_hbm, dst_hbm, zeros_hbm, out_hbm,
                       sidx, didx, rows, agg):
        cid = lax.axis_index("c")
        sid = lax.axis_index("s")
        wid = sid * NCORES + cid
        pltpu.sync_copy(zeros_hbm.at[pl.ds(sid * ROWS_PER_SUB, ROWS_PER_SUB)],
                        agg.at[pl.ds(sid * ROWS_PER_SUB, ROWS_PER_SUB)])
        plsc.subcore_barrier()

        @pl.loop(0, CHUNKS_PER_W)
        def _(i):
            base = (wid * CHUNKS_PER_W + i) * CHUNK
            pltpu.sync_copy(src_hbm.at[pl.ds(base, CHUNK)], sidx)
            pltpu.sync_copy(dst_hbm.at[pl.ds(base, CHUNK)], didx)
            pltpu.sync_copy(hw_hbm.at[sidx], rows)
            pltpu.sync_copy(rows, agg.at[didx], add=True)

        plsc.subcore_barrier()
        pltpu.sync_copy(agg.at[pl.ds(sid * ROWS_PER_SUB, ROWS_PER_SUB)],
                        out_hbm.at[cid, pl.ds(sid * ROWS_PER_SUB, ROWS_PER_SUB)])

    return seg_sum_kernel(hw, src, dst, zeros)


def _tc_encode(xp, WeT, be, Wl0T, Ws0T, bs0):
    def body(x_ref, weT, be_ref, wlT, wsT, bs_ref, m0_ref, s0b_ref):
        h = jnp.dot(x_ref[...], weT[...], preferred_element_type=jnp.float32)
        h = h + be_ref[...]
        m0_ref[...] = jnp.dot(h, wlT[...], preferred_element_type=jnp.float32)
        s0b_ref[...] = jnp.dot(h, wsT[...],
                               preferred_element_type=jnp.float32) + bs_ref[...]

    w_spec = pl.BlockSpec((H, H), lambda i: (0, 0))
    b_spec = pl.BlockSpec((1, H), lambda i: (0, 0))
    row_spec = pl.BlockSpec((BLK, H), lambda i: (i, 0))
    return pl.pallas_call(
        body,
        grid=(GRID,),
        in_specs=[row_spec, w_spec, b_spec, w_spec, w_spec, b_spec],
        out_specs=[row_spec, row_spec],
        out_shape=[jax.ShapeDtypeStruct((NPAD, H), jnp.float32)] * 2,
    )(xp, WeT, be, Wl0T, Ws0T, bs0)


def _tc_spike_mid(agg0, self0b, Wl1T, Ws1T, bs1):
    def body(a_ref, sb_ref, wlT, wsT, bs_ref, m1_ref, s1b_ref, cnt_ref):
        i = pl.program_id(0)
        cur = a_ref[0] + a_ref[1] + sb_ref[...]
        row = lax.broadcasted_iota(jnp.int32, (BLK, H), 0) + i * BLK
        s = jnp.where((cur >= 1.0) & (row < N), 1.0, 0.0)
        m1_ref[...] = jnp.dot(s, wlT[...], preferred_element_type=jnp.float32)
        s1b_ref[...] = jnp.dot(s, wsT[...],
                               preferred_element_type=jnp.float32) + bs_ref[...]

        @pl.when(i == 0)
        def _():
            cnt_ref[...] = jnp.zeros_like(cnt_ref)

        cnt_ref[...] += jnp.sum(s).reshape(1, 1)

    agg_spec = pl.BlockSpec((NCORES, BLK, H), lambda i: (0, i, 0))
    row_spec = pl.BlockSpec((BLK, H), lambda i: (i, 0))
    w_spec = pl.BlockSpec((H, H), lambda i: (0, 0))
    b_spec = pl.BlockSpec((1, H), lambda i: (0, 0))
    return pl.pallas_call(
        body,
        grid=(GRID,),
        in_specs=[agg_spec, row_spec, w_spec, w_spec, b_spec],
        out_specs=[row_spec, row_spec, pl.BlockSpec((1, 1), lambda i: (0, 0))],
        out_shape=[
            jax.ShapeDtypeStruct((NPAD, H), jnp.float32),
            jax.ShapeDtypeStruct((NPAD, H), jnp.float32),
            jax.ShapeDtypeStruct((1, 1), jnp.float32),
        ],
    )(agg0, self0b, Wl1T, Ws1T, bs1)


def _tc_heads(agg1, self1b, Wn1T, bn1, wn2, bn2, Wc1T, bc1, Wc2Tp, bc2p):
    def body(a_ref, sb_ref, wn1T, bn1_ref, wn2_ref, bn2_ref, wc1T, bc1_ref,
             wc2T, bc2_ref, h_ref, np_ref, cnt_ref, gf_ref, gl_ref):
        i = pl.program_id(0)
        cur = a_ref[0] + a_ref[1] + sb_ref[...]
        row = lax.broadcasted_iota(jnp.int32, (BLK, H), 0) + i * BLK
        h = jnp.where((cur >= 1.0) & (row < N), 1.0, 0.0)
        h_ref[...] = h
        nh = jnp.dot(h, wn1T[...], preferred_element_type=jnp.float32)
        nh = jnp.maximum(nh + bn1_ref[...], 0.0)
        logit = jnp.sum(nh * wn2_ref[...], axis=1, keepdims=True) + bn2_ref[0, 0]
        np_ref[...] = jnp.where(
            logit >= 0.0,
            1.0 / (1.0 + jnp.exp(-logit)),
            jnp.exp(logit) / (1.0 + jnp.exp(logit)),
        )

        @pl.when(i == 0)
        def _():
            cnt_ref[...] = jnp.zeros_like(cnt_ref)
            gf_ref[...] = jnp.zeros_like(gf_ref)

        cnt_ref[...] += jnp.sum(h).reshape(1, 1)
        gf_ref[...] += jnp.sum(h, axis=0, keepdims=True)

        @pl.when(i == GRID - 1)
        def _():
            gf = gf_ref[...] / 10000.0
            z = jnp.dot(gf, wc1T[...], preferred_element_type=jnp.float32)
            z = jnp.maximum(z + bc1_ref[...], 0.0)
            gl_ref[...] = jnp.dot(z, wc2T[...],
                                  preferred_element_type=jnp.float32) + bc2_ref[...]

    agg_spec = pl.BlockSpec((NCORES, BLK, H), lambda i: (0, i, 0))
    row_spec = pl.BlockSpec((BLK, H), lambda i: (i, 0))
    fixed = lambda shape: pl.BlockSpec(shape, lambda i: tuple(0 for _ in shape))
    return pl.pallas_call(
        body,
        grid=(GRID,),
        in_specs=[agg_spec, row_spec,
                  fixed((H, H // 2)), fixed((1, H // 2)),
                  fixed((1, H // 2)), fixed((1, 1)),
                  fixed((H, H // 2)), fixed((1, H // 2)),
                  fixed((H // 2, H)), fixed((1, H))],
        out_specs=[row_spec, pl.BlockSpec((BLK, 1), lambda i: (i, 0)),
                   fixed((1, 1)), fixed((1, H)), fixed((1, H))],
        out_shape=[
            jax.ShapeDtypeStruct((NPAD, H), jnp.float32),
            jax.ShapeDtypeStruct((NPAD, 1), jnp.float32),
            jax.ShapeDtypeStruct((1, 1), jnp.float32),
            jax.ShapeDtypeStruct((1, H), jnp.float32),
            jax.ShapeDtypeStruct((1, H), jnp.float32),
        ],
    )(agg1, self1b, Wn1T, bn1, wn2, bn2, Wc1T, bc1, Wc2Tp, bc2p)


def kernel(x, edge_index, W_enc, b_enc, W_lin0, W_self0, b_self0,
           W_lin1, W_self1, b_self1, Wn1, bn1, Wn2, bn2,
           Wc1, bc1, Wc2, bc2):
    f32 = jnp.float32
    xp = jnp.pad(x, ((0, NPAD - N), (0, 0)))

    src = edge_index[0]
    dst = edge_index[1]
    pad_n = EPAD - E
    pad_i = jnp.arange(pad_n, dtype=jnp.int32)
    src_p = jnp.concatenate([src, pad_i % N])
    dst_p = jnp.concatenate([dst, N + pad_i % (NPAD - N)])
    zeros = jnp.zeros((NPAD, H), f32)

    m0, self0b = _tc_encode(
        xp, W_enc.T, b_enc.reshape(1, H), W_lin0.T, W_self0.T,
        b_self0.reshape(1, H))
    agg0 = _sc_segment_sum(m0, src_p, dst_p, zeros)
    m1, self1b, cnt0 = _tc_spike_mid(
        agg0, self0b, W_lin1.T, W_self1.T, b_self1.reshape(1, H))
    agg1 = _sc_segment_sum(m1, src_p, dst_p, zeros)
    hp, npr, cnt1, _gf, gl = _tc_heads(
        agg1, self1b, Wn1.T, bn1.reshape(1, H // 2), Wn2, bn2.reshape(1, 1),
        Wc1.T, bc1.reshape(1, H // 2),
        jnp.pad(Wc2.T, ((0, 0), (0, H - 2))), jnp.pad(bc2, (0, H - 2)).reshape(1, H))

    global_logits = gl[:, :2]
    node_probs = npr[:N]
    h = hp[:N]
    return (global_logits, node_probs, h, cnt0[0, 0], cnt1[0, 0])


# speedup vs baseline: 10.2630x; 1.8619x over previous
import functools

import jax
import jax.numpy as jnp
from jax import lax
from jax.experimental import pallas as pl
from jax.experimental.pallas import tpu as pltpu
from jax.experimental.pallas import tpu_sc as plsc

N = 10000
NPAD = 10240
E = 320000
F = 128
H = 128
CHUNK = 128
NCORES = 2
NSUB = 16
NW = NCORES * NSUB
CHUNKS_PER_W = 80
EPAD = CHUNKS_PER_W * NW * CHUNK
PHASES = 2
CPP = CHUNKS_PER_W // PHASES
NSC = 10112
ROWS_PER_SUB = NSC // NSUB

BLK = 1024
GRID = NPAD // BLK

_sc_mesh = plsc.VectorSubcoreMesh(core_axis_name="c", subcore_axis_name="s")


def _sc_segment_sum(hw, src, dst, zeros):
    @functools.partial(
        pl.kernel,
        out_type=jax.ShapeDtypeStruct((NCORES, NPAD, H), jnp.float32),
        mesh=_sc_mesh,
        scratch_types=[
            pltpu.VMEM((CPP, CHUNK), jnp.int32),
            pltpu.VMEM((CPP, CHUNK), jnp.int32),
            pltpu.VMEM((CHUNK, H), jnp.float32),
            pltpu.VMEM((CHUNK, H), jnp.float32),
            pltpu.VMEM_SHARED((NSC, H), jnp.float32),
            pltpu.SemaphoreType.DMA,
            pltpu.SemaphoreType.DMA,
        ],
    )
    def seg_sum_kernel(hw_hbm, src_hbm, dst_hbm, zeros_hbm, out_hbm,
                       sidx, didx, rows0, rows1, agg, sem0, sem1):
        cid = lax.axis_index("c")
        sid = lax.axis_index("s")
        wid = sid * NCORES + cid
        pltpu.sync_copy(zeros_hbm.at[pl.ds(sid * ROWS_PER_SUB, ROWS_PER_SUB)],
                        agg.at[pl.ds(sid * ROWS_PER_SUB, ROWS_PER_SUB)])
        plsc.subcore_barrier()

        for ph in range(PHASES):
            pltpu.sync_copy(src_hbm.at[wid, ph], sidx)
            pltpu.sync_copy(dst_hbm.at[wid, ph], didx)
            pltpu.async_copy(hw_hbm.at[sidx.at[0]], rows0, sem0)

            @pl.loop(0, CPP // 2)
            def _(j):
                i = j * 2
                pltpu.async_copy(hw_hbm.at[sidx.at[i + 1]], rows1, sem1)
                pltpu.make_async_copy(hw_hbm.at[sidx.at[i]], rows0, sem0).wait()
                pltpu.sync_copy(rows0, agg.at[didx.at[i]], add=True)

                @pl.when(j < CPP // 2 - 1)
                def _():
                    pltpu.async_copy(hw_hbm.at[sidx.at[i + 2]], rows0, sem0)

                pltpu.make_async_copy(hw_hbm.at[sidx.at[i + 1]], rows1,
                                      sem1).wait()
                pltpu.sync_copy(rows1, agg.at[didx.at[i + 1]], add=True)

        plsc.subcore_barrier()
        pltpu.sync_copy(agg.at[pl.ds(sid * ROWS_PER_SUB, ROWS_PER_SUB)],
                        out_hbm.at[cid, pl.ds(sid * ROWS_PER_SUB, ROWS_PER_SUB)])

    return seg_sum_kernel(hw, src, dst, zeros)


def _tc_encode(xp, WeT, be, Wl0T, Ws0T, bs0):
    def body(x_ref, weT, be_ref, wlT, wsT, bs_ref, m0_ref, s0b_ref):
        h = jnp.dot(x_ref[...], weT[...], preferred_element_type=jnp.float32)
        h = h + be_ref[...]
        m0_ref[...] = jnp.dot(h, wlT[...], preferred_element_type=jnp.float32)
        s0b_ref[...] = jnp.dot(h, wsT[...],
                               preferred_element_type=jnp.float32) + bs_ref[...]

    w_spec = pl.BlockSpec((H, H), lambda i: (0, 0))
    b_spec = pl.BlockSpec((1, H), lambda i: (0, 0))
    row_spec = pl.BlockSpec((BLK, H), lambda i: (i, 0))
    return pl.pallas_call(
        body,
        grid=(GRID,),
        in_specs=[row_spec, w_spec, b_spec, w_spec, w_spec, b_spec],
        out_specs=[row_spec, row_spec],
        out_shape=[jax.ShapeDtypeStruct((NPAD, H), jnp.float32)] * 2,
    )(xp, WeT, be, Wl0T, Ws0T, bs0)


def _tc_spike_mid(agg0, self0b, Wl1T, Ws1T, bs1):
    def body(a_ref, sb_ref, wlT, wsT, bs_ref, m1_ref, s1b_ref, cnt_ref):
        i = pl.program_id(0)
        cur = a_ref[0] + a_ref[1] + sb_ref[...]
        row = lax.broadcasted_iota(jnp.int32, (BLK, H), 0) + i * BLK
        s = jnp.where((cur >= 1.0) & (row < N), 1.0, 0.0)
        m1_ref[...] = jnp.dot(s, wlT[...], preferred_element_type=jnp.float32)
        s1b_ref[...] = jnp.dot(s, wsT[...],
                               preferred_element_type=jnp.float32) + bs_ref[...]

        @pl.when(i == 0)
        def _():
            cnt_ref[...] = jnp.zeros_like(cnt_ref)

        cnt_ref[...] += jnp.sum(s).reshape(1, 1)

    agg_spec = pl.BlockSpec((NCORES, BLK, H), lambda i: (0, i, 0))
    row_spec = pl.BlockSpec((BLK, H), lambda i: (i, 0))
    w_spec = pl.BlockSpec((H, H), lambda i: (0, 0))
    b_spec = pl.BlockSpec((1, H), lambda i: (0, 0))
    return pl.pallas_call(
        body,
        grid=(GRID,),
        in_specs=[agg_spec, row_spec, w_spec, w_spec, b_spec],
        out_specs=[row_spec, row_spec, pl.BlockSpec((1, 1), lambda i: (0, 0))],
        out_shape=[
            jax.ShapeDtypeStruct((NPAD, H), jnp.float32),
            jax.ShapeDtypeStruct((NPAD, H), jnp.float32),
            jax.ShapeDtypeStruct((1, 1), jnp.float32),
        ],
    )(agg0, self0b, Wl1T, Ws1T, bs1)


def _tc_heads(agg1, self1b, Wn1T, bn1, wn2, bn2, Wc1T, bc1, Wc2Tp, bc2p):
    def body(a_ref, sb_ref, wn1T, bn1_ref, wn2_ref, bn2_ref, wc1T, bc1_ref,
             wc2T, bc2_ref, h_ref, np_ref, cnt_ref, gf_ref, gl_ref):
        i = pl.program_id(0)
        cur = a_ref[0] + a_ref[1] + sb_ref[...]
        row = lax.broadcasted_iota(jnp.int32, (BLK, H), 0) + i * BLK
        h = jnp.where((cur >= 1.0) & (row < N), 1.0, 0.0)
        h_ref[...] = h
        nh = jnp.dot(h, wn1T[...], preferred_element_type=jnp.float32)
        nh = jnp.maximum(nh + bn1_ref[...], 0.0)
        logit = jnp.sum(nh * wn2_ref[...], axis=1, keepdims=True) + bn2_ref[0, 0]
        np_ref[...] = jnp.where(
            logit >= 0.0,
            1.0 / (1.0 + jnp.exp(-logit)),
            jnp.exp(logit) / (1.0 + jnp.exp(logit)),
        )

        @pl.when(i == 0)
        def _():
            cnt_ref[...] = jnp.zeros_like(cnt_ref)
            gf_ref[...] = jnp.zeros_like(gf_ref)

        cnt_ref[...] += jnp.sum(h).reshape(1, 1)
        gf_ref[...] += jnp.sum(h, axis=0, keepdims=True)

        @pl.when(i == GRID - 1)
        def _():
            gf = gf_ref[...] / 10000.0
            z = jnp.dot(gf, wc1T[...], preferred_element_type=jnp.float32)
            z = jnp.maximum(z + bc1_ref[...], 0.0)
            gl_ref[...] = jnp.dot(z, wc2T[...],
                                  preferred_element_type=jnp.float32) + bc2_ref[...]

    agg_spec = pl.BlockSpec((NCORES, BLK, H), lambda i: (0, i, 0))
    row_spec = pl.BlockSpec((BLK, H), lambda i: (i, 0))
    fixed = lambda shape: pl.BlockSpec(shape, lambda i: tuple(0 for _ in shape))
    return pl.pallas_call(
        body,
        grid=(GRID,),
        in_specs=[agg_spec, row_spec,
                  fixed((H, H // 2)), fixed((1, H // 2)),
                  fixed((1, H // 2)), fixed((1, 1)),
                  fixed((H, H // 2)), fixed((1, H // 2)),
                  fixed((H // 2, H)), fixed((1, H))],
        out_specs=[row_spec, pl.BlockSpec((BLK, 1), lambda i: (i, 0)),
                   fixed((1, 1)), fixed((1, H)), fixed((1, H))],
        out_shape=[
            jax.ShapeDtypeStruct((NPAD, H), jnp.float32),
            jax.ShapeDtypeStruct((NPAD, 1), jnp.float32),
            jax.ShapeDtypeStruct((1, 1), jnp.float32),
            jax.ShapeDtypeStruct((1, H), jnp.float32),
            jax.ShapeDtypeStruct((1, H), jnp.float32),
        ],
    )(agg1, self1b, Wn1T, bn1, wn2, bn2, Wc1T, bc1, Wc2Tp, bc2p)


def kernel(x, edge_index, W_enc, b_enc, W_lin0, W_self0, b_self0,
           W_lin1, W_self1, b_self1, Wn1, bn1, Wn2, bn2,
           Wc1, bc1, Wc2, bc2):
    f32 = jnp.float32
    xp = jnp.pad(x, ((0, NPAD - N), (0, 0)))

    src = edge_index[0]
    dst = edge_index[1]
    pad_n = EPAD - E
    pad_i = jnp.arange(pad_n, dtype=jnp.int32)
    src_p = jnp.concatenate([src, pad_i % N]).reshape(NW, PHASES, CPP, CHUNK)
    dst_p = jnp.concatenate([dst, N + pad_i % (NSC - N)]).reshape(
        NW, PHASES, CPP, CHUNK)
    zeros = jnp.zeros((NPAD, H), f32)

    m0, self0b = _tc_encode(
        xp, W_enc.T, b_enc.reshape(1, H), W_lin0.T, W_self0.T,
        b_self0.reshape(1, H))
    agg0 = _sc_segment_sum(m0, src_p, dst_p, zeros)
    m1, self1b, cnt0 = _tc_spike_mid(
        agg0, self0b, W_lin1.T, W_self1.T, b_self1.reshape(1, H))
    agg1 = _sc_segment_sum(m1, src_p, dst_p, zeros)
    hp, npr, cnt1, _gf, gl = _tc_heads(
        agg1, self1b, Wn1.T, bn1.reshape(1, H // 2), Wn2, bn2.reshape(1, 1),
        Wc1.T, bc1.reshape(1, H // 2),
        jnp.pad(Wc2.T, ((0, 0), (0, H - 2))), jnp.pad(bc2, (0, H - 2)).reshape(1, H))

    global_logits = gl[:, :2]
    node_probs = npr[:N]
    h = hp[:N]
    return (global_logits, node_probs, h, cnt0[0, 0], cnt1[0, 0])


# speedup vs baseline: 10.5883x; 1.0317x over previous
import functools

import jax
import jax.numpy as jnp
from jax import lax
from jax.experimental import pallas as pl
from jax.experimental.pallas import tpu as pltpu
from jax.experimental.pallas import tpu_sc as plsc

N = 10000
NPAD = 10240
E = 320000
F = 128
H = 128
CHUNK = 64
NCORES = 2
NSUB = 16
NW = NCORES * NSUB
CHUNKS_PER_W = 160
EPAD = CHUNKS_PER_W * NW * CHUNK
PHASES = 4
CPP = CHUNKS_PER_W // PHASES
NBUF = 4
NSC = 10112
ROWS_PER_SUB = NSC // NSUB

BLK = 1024
GRID = NPAD // BLK

_sc_mesh = plsc.VectorSubcoreMesh(core_axis_name="c", subcore_axis_name="s")


def _sc_segment_sum(hw, src, dst, zeros):
    @functools.partial(
        pl.kernel,
        out_type=jax.ShapeDtypeStruct((NCORES, NPAD, H), jnp.float32),
        mesh=_sc_mesh,
        scratch_types=[
            pltpu.VMEM((CPP, CHUNK), jnp.int32),
            pltpu.VMEM((CPP, CHUNK), jnp.int32),
        ] + [pltpu.VMEM((CHUNK, H), jnp.float32)] * NBUF + [
            pltpu.VMEM_SHARED((NSC, H), jnp.float32),
        ] + [pltpu.SemaphoreType.DMA] * (2 * NBUF),
    )
    def seg_sum_kernel(hw_hbm, src_hbm, dst_hbm, zeros_hbm, out_hbm,
                       sidx, didx, *scr):
        rows = scr[:NBUF]
        agg = scr[NBUF]
        gsem = scr[NBUF + 1:NBUF + 1 + NBUF]
        ssem = scr[NBUF + 1 + NBUF:]
        cid = lax.axis_index("c")
        sid = lax.axis_index("s")
        wid = sid * NCORES + cid
        pltpu.sync_copy(zeros_hbm.at[pl.ds(sid * ROWS_PER_SUB, ROWS_PER_SUB)],
                        agg.at[pl.ds(sid * ROWS_PER_SUB, ROWS_PER_SUB)])
        plsc.subcore_barrier()

        def gather(i, b):
            return pltpu.make_async_copy(hw_hbm.at[sidx.at[i]], rows[b],
                                         gsem[b])

        def scat(i, b):
            return pltpu.make_async_copy(rows[b], agg.at[didx.at[i]], ssem[b])

        for ph in range(PHASES):
            pltpu.sync_copy(src_hbm.at[wid, ph], sidx)
            pltpu.sync_copy(dst_hbm.at[wid, ph], didx)
            for b in range(NBUF - 1):
                gather(b, b).start()

            @pl.loop(0, CPP // NBUF)
            def _(j):
                for k in range(NBUF):
                    i = j * NBUF + k
                    bn = (k + NBUF - 1) % NBUF

                    @pl.when(i + NBUF - 1 < CPP)
                    def _():
                        @pl.when(i >= 1)
                        def _():
                            scat(i - 1, bn).wait()

                        gather(i + NBUF - 1, bn).start()

                    gather(i, k).wait()
                    scat(i, k).start(add=True)

            for k in range(NBUF):
                scat(CPP - NBUF + k, k).wait()

        plsc.subcore_barrier()
        pltpu.sync_copy(agg.at[pl.ds(sid * ROWS_PER_SUB, ROWS_PER_SUB)],
                        out_hbm.at[cid, pl.ds(sid * ROWS_PER_SUB, ROWS_PER_SUB)])

    return seg_sum_kernel(hw, src, dst, zeros)


def _tc_encode(xp, WeT, be, Wl0T, Ws0T, bs0):
    def body(x_ref, weT, be_ref, wlT, wsT, bs_ref, m0_ref, s0b_ref):
        h = jnp.dot(x_ref[...], weT[...], preferred_element_type=jnp.float32)
        h = h + be_ref[...]
        m0_ref[...] = jnp.dot(h, wlT[...], preferred_element_type=jnp.float32)
        s0b_ref[...] = jnp.dot(h, wsT[...],
                               preferred_element_type=jnp.float32) + bs_ref[...]

    w_spec = pl.BlockSpec((H, H), lambda i: (0, 0))
    b_spec = pl.BlockSpec((1, H), lambda i: (0, 0))
    row_spec = pl.BlockSpec((BLK, H), lambda i: (i, 0))
    return pl.pallas_call(
        body,
        grid=(GRID,),
        in_specs=[row_spec, w_spec, b_spec, w_spec, w_spec, b_spec],
        out_specs=[row_spec, row_spec],
        out_shape=[jax.ShapeDtypeStruct((NPAD, H), jnp.float32)] * 2,
    )(xp, WeT, be, Wl0T, Ws0T, bs0)


def _tc_spike_mid(agg0, self0b, Wl1T, Ws1T, bs1):
    def body(a_ref, sb_ref, wlT, wsT, bs_ref, m1_ref, s1b_ref, cnt_ref):
        i = pl.program_id(0)
        cur = a_ref[0] + a_ref[1] + sb_ref[...]
        row = lax.broadcasted_iota(jnp.int32, (BLK, H), 0) + i * BLK
        s = jnp.where((cur >= 1.0) & (row < N), 1.0, 0.0)
        m1_ref[...] = jnp.dot(s, wlT[...], preferred_element_type=jnp.float32)
        s1b_ref[...] = jnp.dot(s, wsT[...],
                               preferred_element_type=jnp.float32) + bs_ref[...]

        @pl.when(i == 0)
        def _():
            cnt_ref[...] = jnp.zeros_like(cnt_ref)

        cnt_ref[...] += jnp.sum(s).reshape(1, 1)

    agg_spec = pl.BlockSpec((NCORES, BLK, H), lambda i: (0, i, 0))
    row_spec = pl.BlockSpec((BLK, H), lambda i: (i, 0))
    w_spec = pl.BlockSpec((H, H), lambda i: (0, 0))
    b_spec = pl.BlockSpec((1, H), lambda i: (0, 0))
    return pl.pallas_call(
        body,
        grid=(GRID,),
        in_specs=[agg_spec, row_spec, w_spec, w_spec, b_spec],
        out_specs=[row_spec, row_spec, pl.BlockSpec((1, 1), lambda i: (0, 0))],
        out_shape=[
            jax.ShapeDtypeStruct((NPAD, H), jnp.float32),
            jax.ShapeDtypeStruct((NPAD, H), jnp.float32),
            jax.ShapeDtypeStruct((1, 1), jnp.float32),
        ],
    )(agg0, self0b, Wl1T, Ws1T, bs1)


def _tc_heads(agg1, self1b, Wn1T, bn1, wn2, bn2, Wc1T, bc1, Wc2Tp, bc2p):
    def body(a_ref, sb_ref, wn1T, bn1_ref, wn2_ref, bn2_ref, wc1T, bc1_ref,
             wc2T, bc2_ref, h_ref, np_ref, cnt_ref, gf_ref, gl_ref):
        i = pl.program_id(0)
        cur = a_ref[0] + a_ref[1] + sb_ref[...]
        row = lax.broadcasted_iota(jnp.int32, (BLK, H), 0) + i * BLK
        h = jnp.where((cur >= 1.0) & (row < N), 1.0, 0.0)
        h_ref[...] = h
        nh = jnp.dot(h, wn1T[...], preferred_element_type=jnp.float32)
        nh = jnp.maximum(nh + bn1_ref[...], 0.0)
        logit = jnp.sum(nh * wn2_ref[...], axis=1, keepdims=True) + bn2_ref[0, 0]
        np_ref[...] = jnp.where(
            logit >= 0.0,
            1.0 / (1.0 + jnp.exp(-logit)),
            jnp.exp(logit) / (1.0 + jnp.exp(logit)),
        )

        @pl.when(i == 0)
        def _():
            cnt_ref[...] = jnp.zeros_like(cnt_ref)
            gf_ref[...] = jnp.zeros_like(gf_ref)

        cnt_ref[...] += jnp.sum(h).reshape(1, 1)
        gf_ref[...] += jnp.sum(h, axis=0, keepdims=True)

        @pl.when(i == GRID - 1)
        def _():
            gf = gf_ref[...] / 10000.0
            z = jnp.dot(gf, wc1T[...], preferred_element_type=jnp.float32)
            z = jnp.maximum(z + bc1_ref[...], 0.0)
            gl_ref[...] = jnp.dot(z, wc2T[...],
                                  preferred_element_type=jnp.float32) + bc2_ref[...]

    agg_spec = pl.BlockSpec((NCORES, BLK, H), lambda i: (0, i, 0))
    row_spec = pl.BlockSpec((BLK, H), lambda i: (i, 0))
    fixed = lambda shape: pl.BlockSpec(shape, lambda i: tuple(0 for _ in shape))
    return pl.pallas_call(
        body,
        grid=(GRID,),
        in_specs=[agg_spec, row_spec,
                  fixed((H, H // 2)), fixed((1, H // 2)),
                  fixed((1, H // 2)), fixed((1, 1)),
                  fixed((H, H // 2)), fixed((1, H // 2)),
                  fixed((H // 2, H)), fixed((1, H))],
        out_specs=[row_spec, pl.BlockSpec((BLK, 1), lambda i: (i, 0)),
                   fixed((1, 1)), fixed((1, H)), fixed((1, H))],
        out_shape=[
            jax.ShapeDtypeStruct((NPAD, H), jnp.float32),
            jax.ShapeDtypeStruct((NPAD, 1), jnp.float32),
            jax.ShapeDtypeStruct((1, 1), jnp.float32),
            jax.ShapeDtypeStruct((1, H), jnp.float32),
            jax.ShapeDtypeStruct((1, H), jnp.float32),
        ],
    )(agg1, self1b, Wn1T, bn1, wn2, bn2, Wc1T, bc1, Wc2Tp, bc2p)


def kernel(x, edge_index, W_enc, b_enc, W_lin0, W_self0, b_self0,
           W_lin1, W_self1, b_self1, Wn1, bn1, Wn2, bn2,
           Wc1, bc1, Wc2, bc2):
    f32 = jnp.float32
    xp = jnp.pad(x, ((0, NPAD - N), (0, 0)))

    src = edge_index[0]
    dst = edge_index[1]
    pad_n = EPAD - E
    pad_i = jnp.arange(pad_n, dtype=jnp.int32)
    src_p = jnp.concatenate([src, pad_i % N]).reshape(NW, PHASES, CPP, CHUNK)
    dst_p = jnp.concatenate([dst, N + pad_i % (NSC - N)]).reshape(
        NW, PHASES, CPP, CHUNK)
    zeros = jnp.zeros((NPAD, H), f32)

    m0, self0b = _tc_encode(
        xp, W_enc.T, b_enc.reshape(1, H), W_lin0.T, W_self0.T,
        b_self0.reshape(1, H))
    agg0 = _sc_segment_sum(m0, src_p, dst_p, zeros)
    m1, self1b, cnt0 = _tc_spike_mid(
        agg0, self0b, W_lin1.T, W_self1.T, b_self1.reshape(1, H))
    agg1 = _sc_segment_sum(m1, src_p, dst_p, zeros)
    hp, npr, cnt1, _gf, gl = _tc_heads(
        agg1, self1b, Wn1.T, bn1.reshape(1, H // 2), Wn2, bn2.reshape(1, 1),
        Wc1.T, bc1.reshape(1, H // 2),
        jnp.pad(Wc2.T, ((0, 0), (0, H - 2))), jnp.pad(bc2, (0, H - 2)).reshape(1, H))

    global_logits = gl[:, :2]
    node_probs = npr[:N]
    h = hp[:N]
    return (global_logits, node_probs, h, cnt0[0, 0], cnt1[0, 0])


# speedup vs baseline: 12.2800x; 1.1598x over previous
import functools

import jax
import jax.numpy as jnp
from jax import lax
from jax.experimental import pallas as pl
from jax.experimental.pallas import tpu as pltpu
from jax.experimental.pallas import tpu_sc as plsc

N = 10000
NPAD = 10240
E = 320000
F = 128
H = 128
CHUNK = 64
NCORES = 2
NSUB = 16
NW = NCORES * NSUB
CHUNKS_PER_W = 160
EPAD = CHUNKS_PER_W * NW * CHUNK
PHASES = 4
CPP = CHUNKS_PER_W // PHASES
NBUF = 4
NSC = 10112
ROWS_PER_SUB = NSC // NSUB

BLK = 1024
GRID = NPAD // BLK

_sc_mesh = plsc.VectorSubcoreMesh(core_axis_name="c", subcore_axis_name="s")


def _sc_segment_sum(hw, src, dst, zeros, width, dtype, chunk, phases):
    cpw = EPAD // (NW * chunk)
    cpp = cpw // phases

    @functools.partial(
        pl.kernel,
        out_type=jax.ShapeDtypeStruct((NCORES, NPAD, width), dtype),
        mesh=_sc_mesh,
        compiler_params=pltpu.CompilerParams(
            use_tc_tiling_on_sc=(width == H)),
        scratch_types=[
            pltpu.VMEM((cpp, chunk), jnp.int32),
            pltpu.VMEM((cpp, chunk), jnp.int32),
        ] + [pltpu.VMEM((chunk, width), dtype)] * NBUF + [
            pltpu.VMEM_SHARED((NSC, width), dtype),
        ] + [pltpu.SemaphoreType.DMA] * (2 * NBUF),
    )
    def seg_sum_kernel(hw_hbm, src_hbm, dst_hbm, zeros_hbm, out_hbm,
                       sidx, didx, *scr):
        rows = scr[:NBUF]
        agg = scr[NBUF]
        gsem = scr[NBUF + 1:NBUF + 1 + NBUF]
        ssem = scr[NBUF + 1 + NBUF:]
        cid = lax.axis_index("c")
        sid = lax.axis_index("s")
        wid = sid * NCORES + cid
        pltpu.sync_copy(zeros_hbm.at[pl.ds(sid * ROWS_PER_SUB, ROWS_PER_SUB)],
                        agg.at[pl.ds(sid * ROWS_PER_SUB, ROWS_PER_SUB)])
        plsc.subcore_barrier()

        def gather(i, b):
            return pltpu.make_async_copy(hw_hbm.at[sidx.at[i]], rows[b],
                                         gsem[b])

        def scat(i, b):
            return pltpu.make_async_copy(rows[b], agg.at[didx.at[i]], ssem[b])

        for ph in range(phases):
            pltpu.sync_copy(src_hbm.at[wid, ph], sidx)
            pltpu.sync_copy(dst_hbm.at[wid, ph], didx)
            for b in range(NBUF - 1):
                gather(b, b).start()

            @pl.loop(0, cpp // NBUF)
            def _(j):
                for k in range(NBUF):
                    i = j * NBUF + k
                    bn = (k + NBUF - 1) % NBUF

                    @pl.when(i + NBUF - 1 < cpp)
                    def _():
                        @pl.when(i >= 1)
                        def _():
                            scat(i - 1, bn).wait()

                        gather(i + NBUF - 1, bn).start()

                    gather(i, k).wait()
                    scat(i, k).start(add=True)

            for k in range(NBUF):
                scat(cpp - NBUF + k, k).wait()

        plsc.subcore_barrier()
        pltpu.sync_copy(agg.at[pl.ds(sid * ROWS_PER_SUB, ROWS_PER_SUB)],
                        out_hbm.at[cid, pl.ds(sid * ROWS_PER_SUB, ROWS_PER_SUB)])

    return seg_sum_kernel(hw, src, dst, zeros)


def _tc_encode(xp, WeT, be, Wl0T, Ws0T, bs0):
    def body(x_ref, weT, be_ref, wlT, wsT, bs_ref, m0_ref, s0b_ref):
        h = jnp.dot(x_ref[...], weT[...], preferred_element_type=jnp.float32)
        h = h + be_ref[...]
        m0_ref[...] = jnp.dot(h, wlT[...], preferred_element_type=jnp.float32)
        s0b_ref[...] = jnp.dot(h, wsT[...],
                               preferred_element_type=jnp.float32) + bs_ref[...]

    w_spec = pl.BlockSpec((H, H), lambda i: (0, 0))
    b_spec = pl.BlockSpec((1, H), lambda i: (0, 0))
    row_spec = pl.BlockSpec((BLK, H), lambda i: (i, 0))
    return pl.pallas_call(
        body,
        grid=(GRID,),
        in_specs=[row_spec, w_spec, b_spec, w_spec, w_spec, b_spec],
        out_specs=[row_spec, row_spec],
        out_shape=[jax.ShapeDtypeStruct((NPAD, H), jnp.float32)] * 2,
    )(xp, WeT, be, Wl0T, Ws0T, bs0)


def _tc_spike_mid(agg0, self0b, Ws1T, bs1):
    def body(a_ref, sb_ref, wsT, bs_ref, s8_ref, s1b_ref, cnt_ref):
        i = pl.program_id(0)
        cur = a_ref[0] + a_ref[1] + sb_ref[...]
        row = lax.broadcasted_iota(jnp.int32, (BLK, H), 0) + i * BLK
        s = jnp.where((cur >= 1.0) & (row < N), 1.0, 0.0)
        s8_ref[...] = s.astype(jnp.uint8)
        s1b_ref[...] = jnp.dot(s, wsT[...],
                               preferred_element_type=jnp.float32) + bs_ref[...]

        @pl.when(i == 0)
        def _():
            cnt_ref[...] = jnp.zeros_like(cnt_ref)

        cnt_ref[...] += jnp.sum(s).reshape(1, 1)

    agg_spec = pl.BlockSpec((NCORES, BLK, H), lambda i: (0, i, 0))
    row_spec = pl.BlockSpec((BLK, H), lambda i: (i, 0))
    w_spec = pl.BlockSpec((H, H), lambda i: (0, 0))
    b_spec = pl.BlockSpec((1, H), lambda i: (0, 0))
    return pl.pallas_call(
        body,
        grid=(GRID,),
        in_specs=[agg_spec, row_spec, w_spec, b_spec],
        out_specs=[row_spec, row_spec, pl.BlockSpec((1, 1), lambda i: (0, 0))],
        out_shape=[
            jax.ShapeDtypeStruct((NPAD, H), jnp.uint8),
            jax.ShapeDtypeStruct((NPAD, H), jnp.float32),
            jax.ShapeDtypeStruct((1, 1), jnp.float32),
        ],
    )(agg0, self0b, Ws1T, bs1)


def _tc_heads(cnt1p, self1b, Wl1Tp, Wn1T, bn1, wn2, bn2, Wc1T, bc1, Wc2Tp,
              bc2p):
    def body(a_ref, sb_ref, wl1T, wn1T, bn1_ref, wn2_ref, bn2_ref, wc1T,
             bc1_ref, wc2T, bc2_ref, h_ref, np_ref, cnt_ref, gf_ref, gl_ref):
        i = pl.program_id(0)
        wa, wb = a_ref[0], a_ref[1]
        cntf = jnp.concatenate(
            [(((wa >> (8 * k)) & 0xFF) + ((wb >> (8 * k)) & 0xFF)
              ).astype(jnp.float32) for k in range(4)],
            axis=1)
        cur = jnp.dot(cntf, wl1T[...],
                      preferred_element_type=jnp.float32) + sb_ref[...]
        row = lax.broadcasted_iota(jnp.int32, (BLK, H), 0) + i * BLK
        h = jnp.where((cur >= 1.0) & (row < N), 1.0, 0.0)
        h_ref[...] = h
        nh = jnp.dot(h, wn1T[...], preferred_element_type=jnp.float32)
        nh = jnp.maximum(nh + bn1_ref[...], 0.0)
        logit = jnp.sum(nh * wn2_ref[...], axis=1, keepdims=True) + bn2_ref[0, 0]
        np_ref[...] = jnp.where(
            logit >= 0.0,
            1.0 / (1.0 + jnp.exp(-logit)),
            jnp.exp(logit) / (1.0 + jnp.exp(logit)),
        )

        @pl.when(i == 0)
        def _():
            cnt_ref[...] = jnp.zeros_like(cnt_ref)
            gf_ref[...] = jnp.zeros_like(gf_ref)

        cnt_ref[...] += jnp.sum(h).reshape(1, 1)
        gf_ref[...] += jnp.sum(h, axis=0, keepdims=True)

        @pl.when(i == GRID - 1)
        def _():
            gf = gf_ref[...] / 10000.0
            z = jnp.dot(gf, wc1T[...], preferred_element_type=jnp.float32)
            z = jnp.maximum(z + bc1_ref[...], 0.0)
            gl_ref[...] = jnp.dot(z, wc2T[...],
                                  preferred_element_type=jnp.float32) + bc2_ref[...]

    agg_spec = pl.BlockSpec((NCORES, BLK, H // 4), lambda i: (0, i, 0))
    row_spec = pl.BlockSpec((BLK, H), lambda i: (i, 0))
    fixed = lambda shape: pl.BlockSpec(shape, lambda i: tuple(0 for _ in shape))
    return pl.pallas_call(
        body,
        grid=(GRID,),
        in_specs=[agg_spec, row_spec, fixed((H, H)),
                  fixed((H, H // 2)), fixed((1, H // 2)),
                  fixed((1, H // 2)), fixed((1, 1)),
                  fixed((H, H // 2)), fixed((1, H // 2)),
                  fixed((H // 2, H)), fixed((1, H))],
        out_specs=[row_spec, pl.BlockSpec((BLK, 1), lambda i: (i, 0)),
                   fixed((1, 1)), fixed((1, H)), fixed((1, H))],
        out_shape=[
            jax.ShapeDtypeStruct((NPAD, H), jnp.float32),
            jax.ShapeDtypeStruct((NPAD, 1), jnp.float32),
            jax.ShapeDtypeStruct((1, 1), jnp.float32),
            jax.ShapeDtypeStruct((1, H), jnp.float32),
            jax.ShapeDtypeStruct((1, H), jnp.float32),
        ],
    )(cnt1p, self1b, Wl1Tp, Wn1T, bn1, wn2, bn2, Wc1T, bc1, Wc2Tp, bc2p)


def kernel(x, edge_index, W_enc, b_enc, W_lin0, W_self0, b_self0,
           W_lin1, W_self1, b_self1, Wn1, bn1, Wn2, bn2,
           Wc1, bc1, Wc2, bc2):
    f32 = jnp.float32
    xp = jnp.pad(x, ((0, NPAD - N), (0, 0)))

    src = edge_index[0]
    dst = edge_index[1]
    pad_n = EPAD - E
    pad_i = jnp.arange(pad_n, dtype=jnp.int32)
    src_pf = jnp.concatenate([src, pad_i % N])
    dst_pf = jnp.concatenate([dst, N + pad_i % (NSC - N)])
    src_p = src_pf.reshape(NW, PHASES, CPP, CHUNK)
    dst_p = dst_pf.reshape(NW, PHASES, CPP, CHUNK)
    src_p2 = src_pf.reshape(NW, 1, EPAD // NW // 128, 128)
    dst_p2 = dst_pf.reshape(NW, 1, EPAD // NW // 128, 128)
    zeros = jnp.zeros((NSC, H), f32)
    zeros32 = jnp.zeros((NSC, H // 4), jnp.int32)

    m0, self0b = _tc_encode(
        xp, W_enc.T, b_enc.reshape(1, H), W_lin0.T, W_self0.T,
        b_self0.reshape(1, H))
    agg0 = _sc_segment_sum(m0, src_p, dst_p, zeros, H, f32, CHUNK, PHASES)
    s8, self1b, cnt0 = _tc_spike_mid(
        agg0, self0b, W_self1.T, b_self1.reshape(1, H))
    s_pack = jax.lax.bitcast_convert_type(
        s8.reshape(NPAD, H // 4, 4), jnp.int32)
    cnt1p = _sc_segment_sum(s_pack, src_p2, dst_p2, zeros32, H // 4,
                            jnp.int32, 128, 1)
    perm = [4 * j + k for k in range(4) for j in range(H // 4)]
    Wl1Tp = W_lin1.T[jnp.array(perm), :]
    hp, npr, cnt1, _gf, gl = _tc_heads(
        cnt1p, self1b, Wl1Tp, Wn1.T, bn1.reshape(1, H // 2), Wn2,
        bn2.reshape(1, 1), Wc1.T, bc1.reshape(1, H // 2),
        jnp.pad(Wc2.T, ((0, 0), (0, H - 2))), jnp.pad(bc2, (0, H - 2)).reshape(1, H))

    global_logits = gl[:, :2]
    node_probs = npr[:N]
    h = hp[:N]
    return (global_logits, node_probs, h, cnt0[0, 0], cnt1[0, 0])
